# Initial kernel scaffold; baseline (speedup 1.0000x reference)
import jax
import jax.numpy as jnp
from jax.experimental import pallas as pl


def kernel(ini_pred_poly, pred_poly, gt_poly, keyPointsMask):
    raise NotImplementedError("write your pallas kernel here")


# speedup vs baseline: 1.0632x; 1.0632x over previous
import jax
import jax.numpy as jnp
import numpy as np
from jax.experimental import pallas as pl
from jax.experimental.pallas import tpu as pltpu

_B, _NP, _NG, _TIME = 256, 128, 128, 10


def _smooth_l1(d):
    a = jnp.abs(d)
    return jnp.where(a < 1.0, 0.5 * a * a, a - 0.5)


def _dm_kernel(ipx, ipy, ppx, ppy, gxc, gyc, kpmc, out, s1a, s2a, s3a):
    b = pl.program_id(0)

    @pl.when(b == 0)
    def _init():
        s1a[...] = jnp.zeros_like(s1a)
        s2a[...] = jnp.zeros_like(s2a)
        s3a[...] = jnp.zeros_like(s3a)

    ipxr = ipx[0]
    ipyr = ipy[0]
    ppxr = ppx[0]
    ppyr = ppy[0]
    gx = gxc[0]
    gy = gyc[0]
    kpm = kpmc[0]

    gxp = jnp.concatenate([gx[-1:], gx[:-1]], axis=0)
    gyp = jnp.concatenate([gy[-1:], gy[:-1]], axis=0)

    best_d = None
    best_x = None
    best_y = None
    for t in range(_TIME):
        c = np.float32(t) / np.float32(_TIME)
        omc = np.float32(1.0) - c
        ix = gx * c + gxp * omc
        iy = gy * c + gyp * omc
        dx = ix - ipxr
        dy = iy - ipyr
        d = dx * dx + dy * dy
        if t == 0:
            best_d = d
            best_x = jnp.broadcast_to(ix, d.shape)
            best_y = jnp.broadcast_to(iy, d.shape)
        else:
            m = d < best_d
            best_d = jnp.where(m, d, best_d)
            best_x = jnp.where(m, ix, best_x)
            best_y = jnp.where(m, iy, best_y)

    gi = jax.lax.broadcasted_iota(jnp.int32, (_NG, _NP), 0)
    big = jnp.int32(_NG + _NP)
    dmin = jnp.min(best_d, axis=0, keepdims=True)
    gsel = jnp.min(jnp.where(best_d == dmin, gi, big), axis=0,
                   keepdims=True)
    oh = (gi == gsel).astype(jnp.float32)
    nx = jnp.sum(oh * best_x, axis=0, keepdims=True)
    ny = jnp.sum(oh * best_y, axis=0, keepdims=True)
    s1 = jnp.sum(_smooth_l1(ppxr - nx)) + jnp.sum(_smooth_l1(ppyr - ny))

    dxg = gx - ipxr
    dyg = gy - ipyr
    d2 = dxg * dxg + dyg * dyg
    ni = jax.lax.broadcasted_iota(jnp.int32, (_NG, _NP), 1)
    d2min = jnp.min(d2, axis=1, keepdims=True)
    nsel = jnp.min(jnp.where(d2 == d2min, ni, big), axis=1,
                   keepdims=True)
    oh2 = (ni == nsel).astype(jnp.float32)
    spx = jnp.sum(oh2 * ppxr, axis=1, keepdims=True)
    spy = jnp.sum(oh2 * ppyr, axis=1, keepdims=True)
    l2 = _smooth_l1(spx - gx) + _smooth_l1(spy - gy)
    s2 = jnp.sum(l2 * kpm)
    s3 = jnp.sum(kpm)

    s1a[...] = s1a[...] + s1
    s2a[...] = s2a[...] + s2
    s3a[...] = s3a[...] + s3

    loss = 0.5 * (s2a[0, 0] / (2.0 * s3a[0, 0] + 1.0)
                  + s1a[0, 0] / np.float32(_B * _NP * 2))
    out[...] = jnp.broadcast_to(loss, (1, 1))


def _run(ipx3, ipy3, ppx3, ppy3, gxc3, gyc3, kpmc3, interpret=False):
    row_spec = pl.BlockSpec((1, 1, _NP), lambda b: (b, 0, 0))
    col_spec = pl.BlockSpec((1, _NG, 1), lambda b: (b, 0, 0))
    return pl.pallas_call(
        _dm_kernel,
        grid=(_B,),
        in_specs=[row_spec, row_spec, row_spec, row_spec,
                  col_spec, col_spec, col_spec],
        out_specs=pl.BlockSpec((1, 1), lambda b: (0, 0)),
        out_shape=jax.ShapeDtypeStruct((1, 1), jnp.float32),
        scratch_shapes=[pltpu.VMEM((1, 1), jnp.float32)] * 3,
        interpret=interpret,
    )(ipx3, ipy3, ppx3, ppy3, gxc3, gyc3, kpmc3)


def kernel(ini_pred_poly, pred_poly, gt_poly, keyPointsMask):
    ipx3 = ini_pred_poly[:, :, 0].reshape(_B, 1, _NP)
    ipy3 = ini_pred_poly[:, :, 1].reshape(_B, 1, _NP)
    ppx3 = pred_poly[:, :, 0].reshape(_B, 1, _NP)
    ppy3 = pred_poly[:, :, 1].reshape(_B, 1, _NP)
    gxc3 = gt_poly[:, :, 0].reshape(_B, _NG, 1)
    gyc3 = gt_poly[:, :, 1].reshape(_B, _NG, 1)
    kpmc3 = keyPointsMask.reshape(_B, _NG, 1)
    out = _run(ipx3, ipy3, ppx3, ppy3, gxc3, gyc3, kpmc3)
    return out[0, 0]


# speedup vs baseline: 2.1873x; 2.0573x over previous
import jax
import jax.numpy as jnp
import numpy as np
from jax.experimental import pallas as pl
from jax.experimental.pallas import tpu as pltpu

_B, _NP, _NG, _TIME = 256, 128, 128, 10
_BB = 4


def _smooth_l1(d):
    a = jnp.abs(d)
    return jnp.where(a < 1.0, 0.5 * a * a, a - 0.5)


def _one_instance(ipxr, ipyr, ppxr, ppyr, gx, gy, kpm):
    gxp = jnp.concatenate([gx[-1:], gx[:-1]], axis=0)
    gyp = jnp.concatenate([gy[-1:], gy[:-1]], axis=0)

    dxB = gx - ipxr
    dyB = gy - ipyr
    B2 = dxB * dxB + dyB * dyB
    dxA = gxp - ipxr
    dyA = gyp - ipyr
    A2 = jnp.concatenate([B2[-1:, :], B2[:-1, :]], axis=0)
    C2 = dxA * dxB + dyA * dyB

    den = (A2 + B2) - 2.0 * C2
    num = A2 - C2
    cstar = jnp.where(den > 0.0, num / den, 0.0)
    cstar = jnp.clip(cstar, 0.0, 0.9)
    t1 = jnp.minimum(jnp.floor(cstar * 10.0), 9.0)
    t2 = jnp.minimum(t1 + 1.0, 9.0)
    c1 = t1 / 10.0
    c2 = t2 / 10.0
    d1 = A2 - 2.0 * (c1 * num) + (c1 * c1) * den
    d2 = A2 - 2.0 * (c2 * num) + (c2 * c2) * den
    use1 = d1 <= d2
    bd = jnp.where(use1, d1, d2)
    bc = jnp.where(use1, c1, c2)

    gi = jax.lax.broadcasted_iota(jnp.int32, (_NG, _NP), 0)
    big = jnp.int32(_NG + _NP)
    dmin = jnp.min(bd, axis=0, keepdims=True)
    gsel = jnp.min(jnp.where(bd == dmin, gi, big), axis=0,
                   keepdims=True)
    oh = (gi == gsel).astype(jnp.float32)
    csel = jnp.sum(oh * bc, axis=0, keepdims=True)
    gxs = jnp.sum(oh * gx, axis=0, keepdims=True)
    gys = jnp.sum(oh * gy, axis=0, keepdims=True)
    gxps = jnp.sum(oh * gxp, axis=0, keepdims=True)
    gyps = jnp.sum(oh * gyp, axis=0, keepdims=True)
    omc = 1.0 - csel
    nx = gxs * csel + gxps * omc
    ny = gys * csel + gyps * omc
    s1 = jnp.sum(_smooth_l1(ppxr - nx)) + jnp.sum(_smooth_l1(ppyr - ny))

    ni = jax.lax.broadcasted_iota(jnp.int32, (_NG, _NP), 1)
    d2min = jnp.min(B2, axis=1, keepdims=True)
    nsel = jnp.min(jnp.where(B2 == d2min, ni, big), axis=1,
                   keepdims=True)
    oh2 = (ni == nsel).astype(jnp.float32)
    spx = jnp.sum(oh2 * ppxr, axis=1, keepdims=True)
    spy = jnp.sum(oh2 * ppyr, axis=1, keepdims=True)
    l2 = _smooth_l1(spx - gx) + _smooth_l1(spy - gy)
    s2 = jnp.sum(l2 * kpm)
    s3 = jnp.sum(kpm)
    return s1, s2, s3


def _dm_kernel(ipx, ipy, ppx, ppy, gxc, gyc, kpmc, out, s1a, s2a, s3a):
    b = pl.program_id(0)

    @pl.when(b == 0)
    def _init():
        s1a[...] = jnp.zeros_like(s1a)
        s2a[...] = jnp.zeros_like(s2a)
        s3a[...] = jnp.zeros_like(s3a)

    s1 = jnp.float32(0.0)
    s2 = jnp.float32(0.0)
    s3 = jnp.float32(0.0)
    for i in range(_BB):
        a, b2, c = _one_instance(ipx[i], ipy[i], ppx[i], ppy[i],
                                 gxc[i], gyc[i], kpmc[i])
        s1 = s1 + a
        s2 = s2 + b2
        s3 = s3 + c

    s1a[...] = s1a[...] + s1
    s2a[...] = s2a[...] + s2
    s3a[...] = s3a[...] + s3

    loss = 0.5 * (s2a[0, 0] / (2.0 * s3a[0, 0] + 1.0)
                  + s1a[0, 0] / np.float32(_B * _NP * 2))
    out[...] = jnp.broadcast_to(loss, (1, 1))


def _run(ipx3, ipy3, ppx3, ppy3, gxc3, gyc3, kpmc3, interpret=False):
    row_spec = pl.BlockSpec((_BB, 1, _NP), lambda b: (b, 0, 0))
    col_spec = pl.BlockSpec((_BB, _NG, 1), lambda b: (b, 0, 0))
    return pl.pallas_call(
        _dm_kernel,
        grid=(_B // _BB,),
        in_specs=[row_spec, row_spec, row_spec, row_spec,
                  col_spec, col_spec, col_spec],
        out_specs=pl.BlockSpec((1, 1), lambda b: (0, 0)),
        out_shape=jax.ShapeDtypeStruct((1, 1), jnp.float32),
        scratch_shapes=[pltpu.VMEM((1, 1), jnp.float32)] * 3,
        interpret=interpret,
    )(ipx3, ipy3, ppx3, ppy3, gxc3, gyc3, kpmc3)


def kernel(ini_pred_poly, pred_poly, gt_poly, keyPointsMask):
    ipx3 = ini_pred_poly[:, :, 0].reshape(_B, 1, _NP)
    ipy3 = ini_pred_poly[:, :, 1].reshape(_B, 1, _NP)
    ppx3 = pred_poly[:, :, 0].reshape(_B, 1, _NP)
    ppy3 = pred_poly[:, :, 1].reshape(_B, 1, _NP)
    gxc3 = gt_poly[:, :, 0].reshape(_B, _NG, 1)
    gyc3 = gt_poly[:, :, 1].reshape(_B, _NG, 1)
    kpmc3 = keyPointsMask.reshape(_B, _NG, 1)
    out = _run(ipx3, ipy3, ppx3, ppy3, gxc3, gyc3, kpmc3)
    return out[0, 0]


# speedup vs baseline: 3.8371x; 1.7543x over previous
import jax
import jax.numpy as jnp
import numpy as np
from jax.experimental import pallas as pl
from jax.experimental.pallas import tpu as pltpu

_B, _NP, _NG, _TIME = 256, 128, 128, 10
_BB = 8

_DN_TT = (((0,), (0,)), ((), ()))
_DN_NN = (((1,), (0,)), ((), ()))


def _smooth_l1(d):
    a = jnp.abs(d)
    return jnp.where(a < 1.0, 0.5 * a * a, a - 0.5)


def _one_instance(ipxr, ipyr, ppxr, ppyr, gxr, gyr, kpmr):
    f32 = jnp.float32
    ones = jnp.ones((1, _NP), f32)
    zeros = jnp.zeros((1, _NP), f32)
    gxpr = jnp.concatenate([gxr[:, -1:], gxr[:, :-1]], axis=1)
    gypr = jnp.concatenate([gyr[:, -1:], gyr[:, :-1]], axis=1)

    pn = ipxr * ipxr + ipyr * ipyr
    gg = gxr * gxr + gyr * gyr
    gg2 = gxpr * gxr + gypr * gyr
    sxr = gxpr + gxr
    syr = gypr + gyr
    lhs = jnp.concatenate([
        jnp.concatenate([gxr, zeros], axis=1),
        jnp.concatenate([gyr, zeros], axis=1),
        jnp.concatenate([gg, zeros], axis=1),
        jnp.concatenate([ones, zeros], axis=1),
        jnp.concatenate([zeros, gg2], axis=1),
        jnp.concatenate([zeros, sxr], axis=1),
        jnp.concatenate([zeros, syr], axis=1),
        jnp.concatenate([zeros, ones], axis=1),
    ], axis=0)
    rhs = jnp.concatenate([
        -2.0 * ipxr, -2.0 * ipyr, ones, pn,
        ones, -ipxr, -ipyr, pn,
    ], axis=0)
    dd = jax.lax.dot_general(lhs, rhs, _DN_TT,
                             preferred_element_type=f32)
    B2 = dd[:_NG]
    C2 = dd[_NG:]
    A2 = jnp.concatenate([B2[-1:], B2[:-1]], axis=0)

    den = (A2 + B2) - 2.0 * C2
    num = A2 - C2
    num2 = num + num
    cstar = jnp.where(den > 0.0, num * pl.reciprocal(den, approx=True), 0.0)
    t1 = jnp.clip(jnp.floor(cstar * 10.0), 0.0, 9.0)
    t2 = jnp.minimum(t1 + 1.0, 9.0)
    c1 = t1 / 10.0
    c2 = t2 / 10.0
    d1 = A2 - c1 * num2 + (c1 * c1) * den
    d2 = A2 - c2 * num2 + (c2 * c2) * den
    use1 = d1 <= d2
    bd = jnp.where(use1, d1, d2)
    bc = jnp.where(use1, c1, c2)

    gi = jax.lax.broadcasted_iota(jnp.int32, (_NG, _NP), 0)
    big = jnp.int32(_NG + _NP)
    dmin = jnp.min(bd, axis=0, keepdims=True)
    gsel = jnp.min(jnp.where(bd == dmin, gi, big), axis=0, keepdims=True)
    oh = (gi == gsel).astype(f32)
    csel = jnp.sum(oh * bc, axis=0, keepdims=True)
    gx4 = jnp.concatenate([gxr, gxpr, gyr, gypr], axis=0)
    sel4 = jax.lax.dot_general(gx4, oh, _DN_NN,
                               preferred_element_type=f32)
    omc = 1.0 - csel
    nx = sel4[0:1] * csel + sel4[1:2] * omc
    ny = sel4[2:3] * csel + sel4[3:4] * omc
    r1 = _smooth_l1(ppxr - nx) + _smooth_l1(ppyr - ny)

    lhsp = jnp.concatenate([pn, ipxr, ipyr, ones], axis=0)
    rhsg = jnp.concatenate([ones, -2.0 * gxr, -2.0 * gyr, gg], axis=0)
    B2T = jax.lax.dot_general(lhsp, rhsg, _DN_TT,
                              preferred_element_type=f32)
    nit = jax.lax.broadcasted_iota(jnp.int32, (_NP, _NG), 0)
    dminT = jnp.min(B2T, axis=0, keepdims=True)
    nselT = jnp.min(jnp.where(B2T == dminT, nit, big), axis=0, keepdims=True)
    oh2 = (nit == nselT).astype(f32)
    pp2 = jnp.concatenate([ppxr, ppyr], axis=0)
    sp = jax.lax.dot_general(pp2, oh2, _DN_NN,
                             preferred_element_type=f32)
    l2 = _smooth_l1(sp[0:1] - gxr) + _smooth_l1(sp[1:2] - gyr)
    r2 = l2 * kpmr
    return r1, r2


def _dm_kernel(ipx, ipy, ppx, ppy, gxv, gyv, kpmv, out, s1a, s2a, s3a):
    b = pl.program_id(0)

    @pl.when(b == 0)
    def _init():
        s1a[...] = jnp.zeros_like(s1a)
        s2a[...] = jnp.zeros_like(s2a)
        s3a[...] = jnp.zeros_like(s3a)

    s1 = jnp.zeros((1, _NP), jnp.float32)
    s2 = jnp.zeros((1, _NG), jnp.float32)
    s3 = jnp.zeros((1, _NG), jnp.float32)
    for i in range(_BB):
        r1, r2 = _one_instance(ipx[i], ipy[i], ppx[i], ppy[i],
                               gxv[i], gyv[i], kpmv[i])
        s1 = s1 + r1
        s2 = s2 + r2
        s3 = s3 + kpmv[i]

    s1a[...] = s1a[...] + s1
    s2a[...] = s2a[...] + s2
    s3a[...] = s3a[...] + s3

    @pl.when(b == (_B // _BB) - 1)
    def _final():
        t1 = jnp.sum(s1a[...])
        t2 = jnp.sum(s2a[...])
        t3 = jnp.sum(s3a[...])
        loss = 0.5 * (t2 / (2.0 * t3 + 1.0)
                      + t1 / np.float32(_B * _NP * 2))
        out[...] = jnp.broadcast_to(loss, (1, 1))


def _run(ipx3, ipy3, ppx3, ppy3, gx3, gy3, kpm3, interpret=False):
    row_spec = pl.BlockSpec((_BB, 1, _NP), lambda b: (b, 0, 0))
    return pl.pallas_call(
        _dm_kernel,
        grid=(_B // _BB,),
        in_specs=[row_spec] * 7,
        out_specs=pl.BlockSpec((1, 1), lambda b: (0, 0)),
        out_shape=jax.ShapeDtypeStruct((1, 1), jnp.float32),
        scratch_shapes=[pltpu.VMEM((1, _NP), jnp.float32)] * 3,
        interpret=interpret,
    )(ipx3, ipy3, ppx3, ppy3, gx3, gy3, kpm3)


def kernel(ini_pred_poly, pred_poly, gt_poly, keyPointsMask):
    ipx3 = ini_pred_poly[:, :, 0].reshape(_B, 1, _NP)
    ipy3 = ini_pred_poly[:, :, 1].reshape(_B, 1, _NP)
    ppx3 = pred_poly[:, :, 0].reshape(_B, 1, _NP)
    ppy3 = pred_poly[:, :, 1].reshape(_B, 1, _NP)
    gx3 = gt_poly[:, :, 0].reshape(_B, 1, _NG)
    gy3 = gt_poly[:, :, 1].reshape(_B, 1, _NG)
    kpm3 = keyPointsMask.reshape(_B, 1, _NG)
    out = _run(ipx3, ipy3, ppx3, ppy3, gx3, gy3, kpm3)
    return out[0, 0]


# speedup vs baseline: 4.1646x; 1.0854x over previous
import jax
import jax.numpy as jnp
import numpy as np
from jax.experimental import pallas as pl
from jax.experimental.pallas import tpu as pltpu

_B, _NP, _NG, _TIME = 256, 128, 128, 10
_BB = 16

_DN_TT = (((0,), (0,)), ((), ()))
_DN_NN = (((1,), (0,)), ((), ()))


def _smooth_l1(d):
    a = jnp.abs(d)
    return jnp.where(a < 1.0, 0.5 * a * a, a - 0.5)


def _one_instance(ipxr, ipyr, ppxr, ppyr, gxr, gyr, kpmr):
    f32 = jnp.float32
    ones = jnp.ones((1, _NP), f32)
    zeros = jnp.zeros((1, _NP), f32)
    gxpr = jnp.concatenate([gxr[:, -1:], gxr[:, :-1]], axis=1)
    gypr = jnp.concatenate([gyr[:, -1:], gyr[:, :-1]], axis=1)

    pn = ipxr * ipxr + ipyr * ipyr
    gg = gxr * gxr + gyr * gyr
    gg2 = gxpr * gxr + gypr * gyr
    sxr = gxpr + gxr
    syr = gypr + gyr
    lhs = jnp.concatenate([
        jnp.concatenate([gxr, zeros], axis=1),
        jnp.concatenate([gyr, zeros], axis=1),
        jnp.concatenate([gg, zeros], axis=1),
        jnp.concatenate([ones, zeros], axis=1),
        jnp.concatenate([zeros, gg2], axis=1),
        jnp.concatenate([zeros, sxr], axis=1),
        jnp.concatenate([zeros, syr], axis=1),
        jnp.concatenate([zeros, ones], axis=1),
    ], axis=0)
    rhs = jnp.concatenate([
        -2.0 * ipxr, -2.0 * ipyr, ones, pn,
        ones, -ipxr, -ipyr, pn,
    ], axis=0)
    dd = jax.lax.dot_general(lhs, rhs, _DN_TT,
                             preferred_element_type=f32)
    B2 = dd[:_NG]
    C2 = dd[_NG:]
    A2 = jnp.concatenate([B2[-1:], B2[:-1]], axis=0)

    den = (A2 + B2) - 2.0 * C2
    num = A2 - C2
    num2 = num + num
    cstar = jnp.where(den > 0.0, num * pl.reciprocal(den, approx=True), 0.0)
    t1 = jnp.clip(jnp.floor(cstar * 10.0), 0.0, 9.0)
    t2 = jnp.minimum(t1 + 1.0, 9.0)
    c1 = t1 / 10.0
    c2 = t2 / 10.0
    d1 = A2 - c1 * num2 + (c1 * c1) * den
    d2 = A2 - c2 * num2 + (c2 * c2) * den
    use1 = d1 <= d2
    bd = jnp.where(use1, d1, d2)
    bc = jnp.where(use1, c1, c2)

    gi = jax.lax.broadcasted_iota(jnp.int32, (_NG, _NP), 0)
    big = jnp.int32(_NG + _NP)
    dmin = jnp.min(bd, axis=0, keepdims=True)
    gsel = jnp.min(jnp.where(bd == dmin, gi, big), axis=0, keepdims=True)
    oh = (gi == gsel).astype(f32)
    csel = jnp.sum(oh * bc, axis=0, keepdims=True)
    gx4 = jnp.concatenate([gxr, gxpr, gyr, gypr], axis=0)
    sel4 = jax.lax.dot_general(gx4, oh, _DN_NN,
                               preferred_element_type=f32)
    omc = 1.0 - csel
    nx = sel4[0:1] * csel + sel4[1:2] * omc
    ny = sel4[2:3] * csel + sel4[3:4] * omc
    r1 = _smooth_l1(ppxr - nx) + _smooth_l1(ppyr - ny)

    lhsp = jnp.concatenate([pn, ipxr, ipyr, ones], axis=0)
    rhsg = jnp.concatenate([ones, -2.0 * gxr, -2.0 * gyr, gg], axis=0)
    B2T = jax.lax.dot_general(lhsp, rhsg, _DN_TT,
                              preferred_element_type=f32)
    nit = jax.lax.broadcasted_iota(jnp.int32, (_NP, _NG), 0)
    dminT = jnp.min(B2T, axis=0, keepdims=True)
    nselT = jnp.min(jnp.where(B2T == dminT, nit, big), axis=0, keepdims=True)
    oh2 = (nit == nselT).astype(f32)
    pp2 = jnp.concatenate([ppxr, ppyr], axis=0)
    sp = jax.lax.dot_general(pp2, oh2, _DN_NN,
                             preferred_element_type=f32)
    l2 = _smooth_l1(sp[0:1] - gxr) + _smooth_l1(sp[1:2] - gyr)
    r2 = l2 * kpmr
    return r1, r2


def _dm_kernel(ipx, ipy, ppx, ppy, gxv, gyv, kpmv, out, s1a, s2a, s3a):
    b = pl.program_id(0)

    @pl.when(b == 0)
    def _init():
        s1a[...] = jnp.zeros_like(s1a)
        s2a[...] = jnp.zeros_like(s2a)
        s3a[...] = jnp.zeros_like(s3a)

    s1 = jnp.zeros((1, _NP), jnp.float32)
    s2 = jnp.zeros((1, _NG), jnp.float32)
    s3 = jnp.zeros((1, _NG), jnp.float32)
    for i in range(_BB):
        r1, r2 = _one_instance(ipx[i], ipy[i], ppx[i], ppy[i],
                               gxv[i], gyv[i], kpmv[i])
        s1 = s1 + r1
        s2 = s2 + r2
        s3 = s3 + kpmv[i]

    s1a[...] = s1a[...] + s1
    s2a[...] = s2a[...] + s2
    s3a[...] = s3a[...] + s3

    @pl.when(b == (_B // _BB) - 1)
    def _final():
        t1 = jnp.sum(s1a[...])
        t2 = jnp.sum(s2a[...])
        t3 = jnp.sum(s3a[...])
        loss = 0.5 * (t2 / (2.0 * t3 + 1.0)
                      + t1 / np.float32(_B * _NP * 2))
        out[...] = jnp.broadcast_to(loss, (1, 1))


def _run(ipx3, ipy3, ppx3, ppy3, gx3, gy3, kpm3, interpret=False):
    row_spec = pl.BlockSpec((_BB, 1, _NP), lambda b: (b, 0, 0))
    return pl.pallas_call(
        _dm_kernel,
        grid=(_B // _BB,),
        in_specs=[row_spec] * 7,
        out_specs=pl.BlockSpec((1, 1), lambda b: (0, 0)),
        out_shape=jax.ShapeDtypeStruct((1, 1), jnp.float32),
        scratch_shapes=[pltpu.VMEM((1, _NP), jnp.float32)] * 3,
        interpret=interpret,
    )(ipx3, ipy3, ppx3, ppy3, gx3, gy3, kpm3)


def kernel(ini_pred_poly, pred_poly, gt_poly, keyPointsMask):
    ipx3 = ini_pred_poly[:, :, 0].reshape(_B, 1, _NP)
    ipy3 = ini_pred_poly[:, :, 1].reshape(_B, 1, _NP)
    ppx3 = pred_poly[:, :, 0].reshape(_B, 1, _NP)
    ppy3 = pred_poly[:, :, 1].reshape(_B, 1, _NP)
    gx3 = gt_poly[:, :, 0].reshape(_B, 1, _NG)
    gy3 = gt_poly[:, :, 1].reshape(_B, 1, _NG)
    kpm3 = keyPointsMask.reshape(_B, 1, _NG)
    out = _run(ipx3, ipy3, ppx3, ppy3, gx3, gy3, kpm3)
    return out[0, 0]


# speedup vs baseline: 4.5690x; 1.0971x over previous
import jax
import jax.numpy as jnp
import numpy as np
from jax.experimental import pallas as pl
from jax.experimental.pallas import tpu as pltpu

_B, _NP, _NG, _TIME = 256, 128, 128, 10
_BB = 16

_DN_TT = (((0,), (0,)), ((), ()))
_DN_NN = (((1,), (0,)), ((), ()))


def _smooth_l1(d):
    a = jnp.abs(d)
    return jnp.where(a < 1.0, 0.5 * a * a, a - 0.5)


def _one_instance(ipxr, ipyr, ppxr, ppyr, gxr, gyr, kpmr):
    f32 = jnp.float32
    ones = jnp.ones((1, _NP), f32)
    zeros = jnp.zeros((1, _NP), f32)
    gxpr = jnp.concatenate([gxr[:, -1:], gxr[:, :-1]], axis=1)
    gypr = jnp.concatenate([gyr[:, -1:], gyr[:, :-1]], axis=1)

    pn = ipxr * ipxr + ipyr * ipyr
    gg = gxr * gxr + gyr * gyr
    gg2 = gxpr * gxr + gypr * gyr
    sxr = gxpr + gxr
    syr = gypr + gyr
    lhs = jnp.concatenate([
        jnp.concatenate([gxr, zeros], axis=1),
        jnp.concatenate([gyr, zeros], axis=1),
        jnp.concatenate([gg, zeros], axis=1),
        jnp.concatenate([ones, zeros], axis=1),
        jnp.concatenate([zeros, gg2], axis=1),
        jnp.concatenate([zeros, sxr], axis=1),
        jnp.concatenate([zeros, syr], axis=1),
        jnp.concatenate([zeros, ones], axis=1),
    ], axis=0)
    rhs = jnp.concatenate([
        -2.0 * ipxr, -2.0 * ipyr, ones, pn,
        ones, -ipxr, -ipyr, pn,
    ], axis=0)
    dd = jax.lax.dot_general(lhs, rhs, _DN_TT,
                             preferred_element_type=f32)
    B2 = dd[:_NG]
    C2 = dd[_NG:]
    A2 = jnp.concatenate([B2[-1:], B2[:-1]], axis=0)

    den = (A2 + B2) - 2.0 * C2
    num = A2 - C2
    num2 = num + num
    cstar = jnp.where(den > 0.0, num * pl.reciprocal(den, approx=True), 0.0)
    t1 = jnp.clip(jnp.floor(cstar * 10.0), 0.0, 9.0)
    t2 = jnp.minimum(t1 + 1.0, 9.0)
    use1 = den * ((t1 + t2) / 10.0) >= num2
    bc = jnp.where(use1, t1, t2) / 10.0
    bd = A2 - bc * num2 + (bc * bc) * den

    dmin = jnp.min(bd, axis=0, keepdims=True)
    oh = (bd == dmin).astype(f32)
    csel = jnp.sum(oh * bc, axis=0, keepdims=True)
    gx4 = jnp.concatenate([gxr, gxpr, gyr, gypr], axis=0)
    sel4 = jax.lax.dot_general(gx4, oh, _DN_NN,
                               preferred_element_type=f32)
    omc = 1.0 - csel
    nx = sel4[0:1] * csel + sel4[1:2] * omc
    ny = sel4[2:3] * csel + sel4[3:4] * omc
    r1 = _smooth_l1(ppxr - nx) + _smooth_l1(ppyr - ny)

    lhsp = jnp.concatenate([pn, ipxr, ipyr, ones], axis=0)
    rhsg = jnp.concatenate([ones, -2.0 * gxr, -2.0 * gyr, gg], axis=0)
    B2T = jax.lax.dot_general(lhsp, rhsg, _DN_TT,
                              preferred_element_type=f32)
    dminT = jnp.min(B2T, axis=0, keepdims=True)
    oh2 = (B2T == dminT).astype(f32)
    pp2 = jnp.concatenate([ppxr, ppyr], axis=0)
    sp = jax.lax.dot_general(pp2, oh2, _DN_NN,
                             preferred_element_type=f32)
    l2 = _smooth_l1(sp[0:1] - gxr) + _smooth_l1(sp[1:2] - gyr)
    r2 = l2 * kpmr
    return r1, r2


def _dm_kernel(ipx, ipy, ppx, ppy, gxv, gyv, kpmv, out, s1a, s2a, s3a):
    b = pl.program_id(0)

    @pl.when(b == 0)
    def _init():
        s1a[...] = jnp.zeros_like(s1a)
        s2a[...] = jnp.zeros_like(s2a)
        s3a[...] = jnp.zeros_like(s3a)

    s1 = jnp.zeros((1, _NP), jnp.float32)
    s2 = jnp.zeros((1, _NG), jnp.float32)
    s3 = jnp.zeros((1, _NG), jnp.float32)
    for i in range(_BB):
        r1, r2 = _one_instance(ipx[i], ipy[i], ppx[i], ppy[i],
                               gxv[i], gyv[i], kpmv[i])
        s1 = s1 + r1
        s2 = s2 + r2
        s3 = s3 + kpmv[i]

    s1a[...] = s1a[...] + s1
    s2a[...] = s2a[...] + s2
    s3a[...] = s3a[...] + s3

    @pl.when(b == (_B // _BB) - 1)
    def _final():
        t1 = jnp.sum(s1a[...])
        t2 = jnp.sum(s2a[...])
        t3 = jnp.sum(s3a[...])
        loss = 0.5 * (t2 / (2.0 * t3 + 1.0)
                      + t1 / np.float32(_B * _NP * 2))
        out[...] = jnp.broadcast_to(loss, (1, 1))


def _run(ipx3, ipy3, ppx3, ppy3, gx3, gy3, kpm3, interpret=False):
    row_spec = pl.BlockSpec((_BB, 1, _NP), lambda b: (b, 0, 0))
    return pl.pallas_call(
        _dm_kernel,
        grid=(_B // _BB,),
        in_specs=[row_spec] * 7,
        out_specs=pl.BlockSpec((1, 1), lambda b: (0, 0)),
        out_shape=jax.ShapeDtypeStruct((1, 1), jnp.float32),
        scratch_shapes=[pltpu.VMEM((1, _NP), jnp.float32)] * 3,
        interpret=interpret,
    )(ipx3, ipy3, ppx3, ppy3, gx3, gy3, kpm3)


def kernel(ini_pred_poly, pred_poly, gt_poly, keyPointsMask):
    ipx3 = ini_pred_poly[:, :, 0].reshape(_B, 1, _NP)
    ipy3 = ini_pred_poly[:, :, 1].reshape(_B, 1, _NP)
    ppx3 = pred_poly[:, :, 0].reshape(_B, 1, _NP)
    ppy3 = pred_poly[:, :, 1].reshape(_B, 1, _NP)
    gx3 = gt_poly[:, :, 0].reshape(_B, 1, _NG)
    gy3 = gt_poly[:, :, 1].reshape(_B, 1, _NG)
    kpm3 = keyPointsMask.reshape(_B, 1, _NG)
    out = _run(ipx3, ipy3, ppx3, ppy3, gx3, gy3, kpm3)
    return out[0, 0]


# speedup vs baseline: 4.7775x; 1.0456x over previous
import jax
import jax.numpy as jnp
import numpy as np
from jax.experimental import pallas as pl
from jax.experimental.pallas import tpu as pltpu

_B, _NP, _NG, _TIME = 256, 128, 128, 10
_BB = 32

_DN_TT = (((0,), (0,)), ((), ()))
_DN_NN = (((1,), (0,)), ((), ()))


def _smooth_l1(d):
    a = jnp.abs(d)
    return jnp.where(a < 1.0, 0.5 * a * a, a - 0.5)


def _one_instance(ipxr, ipyr, ppxr, ppyr, gxr, gyr, kpmr):
    f32 = jnp.float32
    ones = jnp.ones((1, _NP), f32)
    zeros = jnp.zeros((1, _NP), f32)
    gxpr = jnp.concatenate([gxr[:, -1:], gxr[:, :-1]], axis=1)
    gypr = jnp.concatenate([gyr[:, -1:], gyr[:, :-1]], axis=1)

    pn = ipxr * ipxr + ipyr * ipyr
    gg = gxr * gxr + gyr * gyr
    gg2 = gxpr * gxr + gypr * gyr
    sxr = gxpr + gxr
    syr = gypr + gyr
    lhs = jnp.concatenate([
        jnp.concatenate([gxr, zeros], axis=1),
        jnp.concatenate([gyr, zeros], axis=1),
        jnp.concatenate([gg, zeros], axis=1),
        jnp.concatenate([ones, zeros], axis=1),
        jnp.concatenate([zeros, gg2], axis=1),
        jnp.concatenate([zeros, sxr], axis=1),
        jnp.concatenate([zeros, syr], axis=1),
        jnp.concatenate([zeros, ones], axis=1),
    ], axis=0)
    rhs = jnp.concatenate([
        -2.0 * ipxr, -2.0 * ipyr, ones, pn,
        ones, -ipxr, -ipyr, pn,
    ], axis=0)
    dd = jax.lax.dot_general(lhs, rhs, _DN_TT,
                             preferred_element_type=f32)
    B2 = dd[:_NG]
    C2 = dd[_NG:]
    A2 = jnp.concatenate([B2[-1:], B2[:-1]], axis=0)

    den = (A2 + B2) - 2.0 * C2
    num = A2 - C2
    num2 = num + num
    cstar = jnp.where(den > 0.0, num * pl.reciprocal(den, approx=True), 0.0)
    t1 = jnp.clip(jnp.floor(cstar * 10.0), 0.0, 9.0)
    t2 = jnp.minimum(t1 + 1.0, 9.0)
    use1 = den * ((t1 + t2) / 10.0) >= num2
    bc = jnp.where(use1, t1, t2) / 10.0
    bd = A2 - bc * num2 + (bc * bc) * den

    dmin = jnp.min(bd, axis=0, keepdims=True)
    oh = (bd == dmin).astype(f32)
    csel = jnp.sum(oh * bc, axis=0, keepdims=True)
    gx4 = jnp.concatenate([gxr, gxpr, gyr, gypr], axis=0)
    sel4 = jax.lax.dot_general(gx4, oh, _DN_NN,
                               preferred_element_type=f32)
    omc = 1.0 - csel
    nx = sel4[0:1] * csel + sel4[1:2] * omc
    ny = sel4[2:3] * csel + sel4[3:4] * omc
    r1 = _smooth_l1(ppxr - nx) + _smooth_l1(ppyr - ny)

    lhsp = jnp.concatenate([pn, ipxr, ipyr, ones], axis=0)
    rhsg = jnp.concatenate([ones, -2.0 * gxr, -2.0 * gyr, gg], axis=0)
    B2T = jax.lax.dot_general(lhsp, rhsg, _DN_TT,
                              preferred_element_type=f32)
    dminT = jnp.min(B2T, axis=0, keepdims=True)
    oh2 = (B2T == dminT).astype(f32)
    pp2 = jnp.concatenate([ppxr, ppyr], axis=0)
    sp = jax.lax.dot_general(pp2, oh2, _DN_NN,
                             preferred_element_type=f32)
    l2 = _smooth_l1(sp[0:1] - gxr) + _smooth_l1(sp[1:2] - gyr)
    r2 = l2 * kpmr
    return r1, r2


def _dm_kernel(ipx, ipy, ppx, ppy, gxv, gyv, kpmv, out, s1a, s2a, s3a):
    b = pl.program_id(0)

    @pl.when(b == 0)
    def _init():
        s1a[...] = jnp.zeros_like(s1a)
        s2a[...] = jnp.zeros_like(s2a)
        s3a[...] = jnp.zeros_like(s3a)

    s1 = jnp.zeros((1, _NP), jnp.float32)
    s2 = jnp.zeros((1, _NG), jnp.float32)
    s3 = jnp.zeros((1, _NG), jnp.float32)
    for i in range(_BB):
        r1, r2 = _one_instance(ipx[i], ipy[i], ppx[i], ppy[i],
                               gxv[i], gyv[i], kpmv[i])
        s1 = s1 + r1
        s2 = s2 + r2
        s3 = s3 + kpmv[i]

    s1a[...] = s1a[...] + s1
    s2a[...] = s2a[...] + s2
    s3a[...] = s3a[...] + s3

    @pl.when(b == (_B // _BB) - 1)
    def _final():
        t1 = jnp.sum(s1a[...])
        t2 = jnp.sum(s2a[...])
        t3 = jnp.sum(s3a[...])
        loss = 0.5 * (t2 / (2.0 * t3 + 1.0)
                      + t1 / np.float32(_B * _NP * 2))
        out[...] = jnp.broadcast_to(loss, (1, 1))


def _run(ipx3, ipy3, ppx3, ppy3, gx3, gy3, kpm3, interpret=False):
    row_spec = pl.BlockSpec((_BB, 1, _NP), lambda b: (b, 0, 0))
    return pl.pallas_call(
        _dm_kernel,
        grid=(_B // _BB,),
        in_specs=[row_spec] * 7,
        out_specs=pl.BlockSpec((1, 1), lambda b: (0, 0)),
        out_shape=jax.ShapeDtypeStruct((1, 1), jnp.float32),
        scratch_shapes=[pltpu.VMEM((1, _NP), jnp.float32)] * 3,
        interpret=interpret,
    )(ipx3, ipy3, ppx3, ppy3, gx3, gy3, kpm3)


def kernel(ini_pred_poly, pred_poly, gt_poly, keyPointsMask):
    ipx3 = ini_pred_poly[:, :, 0].reshape(_B, 1, _NP)
    ipy3 = ini_pred_poly[:, :, 1].reshape(_B, 1, _NP)
    ppx3 = pred_poly[:, :, 0].reshape(_B, 1, _NP)
    ppy3 = pred_poly[:, :, 1].reshape(_B, 1, _NP)
    gx3 = gt_poly[:, :, 0].reshape(_B, 1, _NG)
    gy3 = gt_poly[:, :, 1].reshape(_B, 1, _NG)
    kpm3 = keyPointsMask.reshape(_B, 1, _NG)
    out = _run(ipx3, ipy3, ppx3, ppy3, gx3, gy3, kpm3)
    return out[0, 0]


# speedup vs baseline: 5.0512x; 1.0573x over previous
import jax
import jax.numpy as jnp
import numpy as np
from jax.experimental import pallas as pl
from jax.experimental.pallas import tpu as pltpu

_B, _NP, _NG, _TIME = 256, 128, 128, 10
_BB = 32

_DN_TT = (((0,), (0,)), ((), ()))
_DN_NN = (((1,), (0,)), ((), ()))


def _smooth_l1(d):
    a = jnp.abs(d)
    return jnp.where(a < 1.0, 0.5 * a * a, a - 0.5)


def _one_instance(ipxr, ipyr, ppxr, ppyr, gxr, gyr, kpmr):
    f32 = jnp.float32
    ones = jnp.ones((1, _NP), f32)
    zeros = jnp.zeros((1, _NP), f32)
    gxpr = jnp.concatenate([gxr[:, -1:], gxr[:, :-1]], axis=1)
    gypr = jnp.concatenate([gyr[:, -1:], gyr[:, :-1]], axis=1)

    pn = ipxr * ipxr + ipyr * ipyr
    gg = gxr * gxr + gyr * gyr
    gg2 = gxpr * gxr + gypr * gyr
    sxr = gxpr + gxr
    syr = gypr + gyr
    lhs = jnp.concatenate([
        jnp.concatenate([gxr, zeros], axis=1),
        jnp.concatenate([gyr, zeros], axis=1),
        jnp.concatenate([gg, zeros], axis=1),
        jnp.concatenate([ones, zeros], axis=1),
        jnp.concatenate([zeros, gg2], axis=1),
        jnp.concatenate([zeros, sxr], axis=1),
        jnp.concatenate([zeros, syr], axis=1),
        jnp.concatenate([zeros, ones], axis=1),
    ], axis=0)
    rhs = jnp.concatenate([
        -2.0 * ipxr, -2.0 * ipyr, ones, pn,
        ones, -ipxr, -ipyr, pn,
    ], axis=0)
    dd = jax.lax.dot_general(lhs, rhs, _DN_TT,
                             preferred_element_type=f32)
    B2 = dd[:_NG]
    C2 = dd[_NG:]
    A2 = jnp.concatenate([B2[-1:], B2[:-1]], axis=0)

    den = (A2 + B2) - 2.0 * C2
    num = A2 - C2
    num2 = num + num
    cstar = jnp.where(den > 0.0, num * pl.reciprocal(den, approx=True), 0.0)
    bc = jnp.clip(jnp.floor(cstar * 10.0 + 0.5), 0.0, 9.0) / 10.0
    bd = A2 - bc * num2 + (bc * bc) * den

    dmin = jnp.min(bd, axis=0, keepdims=True)
    oh = (bd == dmin).astype(f32)
    csel = jnp.sum(oh * bc, axis=0, keepdims=True)
    gx4 = jnp.concatenate([gxr, gxpr, gyr, gypr], axis=0)
    sel4 = jax.lax.dot_general(gx4, oh, _DN_NN,
                               preferred_element_type=f32)
    omc = 1.0 - csel
    nx = sel4[0:1] * csel + sel4[1:2] * omc
    ny = sel4[2:3] * csel + sel4[3:4] * omc
    r1 = _smooth_l1(ppxr - nx) + _smooth_l1(ppyr - ny)

    lhsp = jnp.concatenate([pn, ipxr, ipyr, ones], axis=0)
    rhsg = jnp.concatenate([ones, -2.0 * gxr, -2.0 * gyr, gg], axis=0)
    B2T = jax.lax.dot_general(lhsp, rhsg, _DN_TT,
                              preferred_element_type=f32)
    dminT = jnp.min(B2T, axis=0, keepdims=True)
    oh2 = (B2T == dminT).astype(f32)
    pp2 = jnp.concatenate([ppxr, ppyr], axis=0)
    sp = jax.lax.dot_general(pp2, oh2, _DN_NN,
                             preferred_element_type=f32)
    l2 = _smooth_l1(sp[0:1] - gxr) + _smooth_l1(sp[1:2] - gyr)
    r2 = l2 * kpmr
    return r1, r2


def _dm_kernel(ipx, ipy, ppx, ppy, gxv, gyv, kpmv, out, s1a, s2a, s3a):
    b = pl.program_id(0)

    @pl.when(b == 0)
    def _init():
        s1a[...] = jnp.zeros_like(s1a)
        s2a[...] = jnp.zeros_like(s2a)
        s3a[...] = jnp.zeros_like(s3a)

    s1 = jnp.zeros((1, _NP), jnp.float32)
    s2 = jnp.zeros((1, _NG), jnp.float32)
    s3 = jnp.zeros((1, _NG), jnp.float32)
    for i in range(_BB):
        r1, r2 = _one_instance(ipx[i], ipy[i], ppx[i], ppy[i],
                               gxv[i], gyv[i], kpmv[i])
        s1 = s1 + r1
        s2 = s2 + r2
        s3 = s3 + kpmv[i]

    s1a[...] = s1a[...] + s1
    s2a[...] = s2a[...] + s2
    s3a[...] = s3a[...] + s3

    @pl.when(b == (_B // _BB) - 1)
    def _final():
        t1 = jnp.sum(s1a[...])
        t2 = jnp.sum(s2a[...])
        t3 = jnp.sum(s3a[...])
        loss = 0.5 * (t2 / (2.0 * t3 + 1.0)
                      + t1 / np.float32(_B * _NP * 2))
        out[...] = jnp.broadcast_to(loss, (1, 1))


def _run(ipx3, ipy3, ppx3, ppy3, gx3, gy3, kpm3, interpret=False):
    row_spec = pl.BlockSpec((_BB, 1, _NP), lambda b: (b, 0, 0))
    return pl.pallas_call(
        _dm_kernel,
        grid=(_B // _BB,),
        in_specs=[row_spec] * 7,
        out_specs=pl.BlockSpec((1, 1), lambda b: (0, 0)),
        out_shape=jax.ShapeDtypeStruct((1, 1), jnp.float32),
        scratch_shapes=[pltpu.VMEM((1, _NP), jnp.float32)] * 3,
        interpret=interpret,
    )(ipx3, ipy3, ppx3, ppy3, gx3, gy3, kpm3)


def kernel(ini_pred_poly, pred_poly, gt_poly, keyPointsMask):
    ipx3 = ini_pred_poly[:, :, 0].reshape(_B, 1, _NP)
    ipy3 = ini_pred_poly[:, :, 1].reshape(_B, 1, _NP)
    ppx3 = pred_poly[:, :, 0].reshape(_B, 1, _NP)
    ppy3 = pred_poly[:, :, 1].reshape(_B, 1, _NP)
    gx3 = gt_poly[:, :, 0].reshape(_B, 1, _NG)
    gy3 = gt_poly[:, :, 1].reshape(_B, 1, _NG)
    kpm3 = keyPointsMask.reshape(_B, 1, _NG)
    out = _run(ipx3, ipy3, ppx3, ppy3, gx3, gy3, kpm3)
    return out[0, 0]


# speedup vs baseline: 5.1694x; 1.0234x over previous
import jax
import jax.numpy as jnp
import numpy as np
from jax.experimental import pallas as pl
from jax.experimental.pallas import tpu as pltpu

_B, _NP, _NG, _TIME = 256, 128, 128, 10
_BB = 64

_DN_TT = (((0,), (0,)), ((), ()))
_DN_NN = (((1,), (0,)), ((), ()))


def _smooth_l1(d):
    a = jnp.abs(d)
    return jnp.where(a < 1.0, 0.5 * a * a, a - 0.5)


def _one_instance(ipxr, ipyr, ppxr, ppyr, gxr, gyr, kpmr):
    f32 = jnp.float32
    ones = jnp.ones((1, _NP), f32)
    zeros = jnp.zeros((1, _NP), f32)
    gxpr = jnp.concatenate([gxr[:, -1:], gxr[:, :-1]], axis=1)
    gypr = jnp.concatenate([gyr[:, -1:], gyr[:, :-1]], axis=1)

    pn = ipxr * ipxr + ipyr * ipyr
    gg = gxr * gxr + gyr * gyr
    gg2 = gxpr * gxr + gypr * gyr
    sxr = gxpr + gxr
    syr = gypr + gyr
    lhs = jnp.concatenate([
        jnp.concatenate([gxr, zeros], axis=1),
        jnp.concatenate([gyr, zeros], axis=1),
        jnp.concatenate([gg, zeros], axis=1),
        jnp.concatenate([ones, zeros], axis=1),
        jnp.concatenate([zeros, gg2], axis=1),
        jnp.concatenate([zeros, sxr], axis=1),
        jnp.concatenate([zeros, syr], axis=1),
        jnp.concatenate([zeros, ones], axis=1),
    ], axis=0)
    rhs = jnp.concatenate([
        -2.0 * ipxr, -2.0 * ipyr, ones, pn,
        ones, -ipxr, -ipyr, pn,
    ], axis=0)
    dd = jax.lax.dot_general(lhs, rhs, _DN_TT,
                             preferred_element_type=f32)
    B2 = dd[:_NG]
    C2 = dd[_NG:]
    A2 = jnp.concatenate([B2[-1:], B2[:-1]], axis=0)

    den = (A2 + B2) - 2.0 * C2
    num = A2 - C2
    num2 = num + num
    cstar = jnp.where(den > 0.0, num * pl.reciprocal(den, approx=False), 0.0)
    bc = jnp.clip(jnp.floor(cstar * 10.0 + 0.5), 0.0, 9.0) / 10.0
    bd = A2 - bc * num2 + (bc * bc) * den

    dmin = jnp.min(bd, axis=0, keepdims=True)
    oh = (bd == dmin).astype(f32)
    csel = jnp.sum(oh * bc, axis=0, keepdims=True)
    gx4 = jnp.concatenate([gxr, gxpr, gyr, gypr], axis=0)
    sel4 = jax.lax.dot_general(gx4, oh, _DN_NN,
                               preferred_element_type=f32)
    omc = 1.0 - csel
    nx = sel4[0:1] * csel + sel4[1:2] * omc
    ny = sel4[2:3] * csel + sel4[3:4] * omc
    r1 = _smooth_l1(ppxr - nx) + _smooth_l1(ppyr - ny)

    lhsp = jnp.concatenate([pn, ipxr, ipyr, ones], axis=0)
    rhsg = jnp.concatenate([ones, -2.0 * gxr, -2.0 * gyr, gg], axis=0)
    B2T = jax.lax.dot_general(lhsp, rhsg, _DN_TT,
                              preferred_element_type=f32)
    dminT = jnp.min(B2T, axis=0, keepdims=True)
    oh2 = (B2T == dminT).astype(f32)
    pp2 = jnp.concatenate([ppxr, ppyr], axis=0)
    sp = jax.lax.dot_general(pp2, oh2, _DN_NN,
                             preferred_element_type=f32)
    l2 = _smooth_l1(sp[0:1] - gxr) + _smooth_l1(sp[1:2] - gyr)
    r2 = l2 * kpmr
    return r1, r2


def _dm_kernel(ipx, ipy, ppx, ppy, gxv, gyv, kpmv, out, s1a, s2a, s3a):
    b = pl.program_id(0)

    @pl.when(b == 0)
    def _init():
        s1a[...] = jnp.zeros_like(s1a)
        s2a[...] = jnp.zeros_like(s2a)
        s3a[...] = jnp.zeros_like(s3a)

    s1 = jnp.zeros((1, _NP), jnp.float32)
    s2 = jnp.zeros((1, _NG), jnp.float32)
    s3 = jnp.zeros((1, _NG), jnp.float32)
    for i in range(_BB):
        r1, r2 = _one_instance(ipx[i], ipy[i], ppx[i], ppy[i],
                               gxv[i], gyv[i], kpmv[i])
        s1 = s1 + r1
        s2 = s2 + r2
        s3 = s3 + kpmv[i]

    s1a[...] = s1a[...] + s1
    s2a[...] = s2a[...] + s2
    s3a[...] = s3a[...] + s3

    @pl.when(b == (_B // _BB) - 1)
    def _final():
        t1 = jnp.sum(s1a[...])
        t2 = jnp.sum(s2a[...])
        t3 = jnp.sum(s3a[...])
        loss = 0.5 * (t2 / (2.0 * t3 + 1.0)
                      + t1 / np.float32(_B * _NP * 2))
        out[...] = jnp.broadcast_to(loss, (1, 1))


def _run(ipx3, ipy3, ppx3, ppy3, gx3, gy3, kpm3, interpret=False):
    row_spec = pl.BlockSpec((_BB, 1, _NP), lambda b: (b, 0, 0))
    return pl.pallas_call(
        _dm_kernel,
        grid=(_B // _BB,),
        in_specs=[row_spec] * 7,
        out_specs=pl.BlockSpec((1, 1), lambda b: (0, 0)),
        out_shape=jax.ShapeDtypeStruct((1, 1), jnp.float32),
        scratch_shapes=[pltpu.VMEM((1, _NP), jnp.float32)] * 3,
        interpret=interpret,
    )(ipx3, ipy3, ppx3, ppy3, gx3, gy3, kpm3)


def kernel(ini_pred_poly, pred_poly, gt_poly, keyPointsMask):
    ipx3 = ini_pred_poly[:, :, 0].reshape(_B, 1, _NP)
    ipy3 = ini_pred_poly[:, :, 1].reshape(_B, 1, _NP)
    ppx3 = pred_poly[:, :, 0].reshape(_B, 1, _NP)
    ppy3 = pred_poly[:, :, 1].reshape(_B, 1, _NP)
    gx3 = gt_poly[:, :, 0].reshape(_B, 1, _NG)
    gy3 = gt_poly[:, :, 1].reshape(_B, 1, _NG)
    kpm3 = keyPointsMask.reshape(_B, 1, _NG)
    out = _run(ipx3, ipy3, ppx3, ppy3, gx3, gy3, kpm3)
    return out[0, 0]


# speedup vs baseline: 5.1715x; 1.0004x over previous
import jax
import jax.numpy as jnp
import numpy as np
from jax.experimental import pallas as pl
from jax.experimental.pallas import tpu as pltpu

_B, _NP, _NG, _TIME = 256, 128, 128, 10
_BB = 64

_DN_TT = (((0,), (0,)), ((), ()))
_DN_NN = (((1,), (0,)), ((), ()))


def _smooth_l1(d):
    a = jnp.abs(d)
    return jnp.where(a < 1.0, 0.5 * a * a, a - 0.5)


def _one_instance(ipxr, ipyr, ppxr, ppyr, gxr, gyr, kpmr):
    f32 = jnp.float32
    ones = jnp.ones((1, _NP), f32)
    zeros = jnp.zeros((1, _NP), f32)
    gxpr = jnp.concatenate([gxr[:, -1:], gxr[:, :-1]], axis=1)
    gypr = jnp.concatenate([gyr[:, -1:], gyr[:, :-1]], axis=1)

    pn = ipxr * ipxr + ipyr * ipyr
    gg = gxr * gxr + gyr * gyr
    gg2 = gxpr * gxr + gypr * gyr
    hxr = 0.5 * (gxpr + gxr)
    hyr = 0.5 * (gypr + gyr)
    lhs = jnp.concatenate([
        jnp.concatenate([gxr, hxr], axis=1),
        jnp.concatenate([gyr, hyr], axis=1),
        jnp.concatenate([gg, gg2], axis=1),
        jnp.concatenate([ones, ones], axis=1),
    ], axis=0)
    rhs = jnp.concatenate([-2.0 * ipxr, -2.0 * ipyr, ones, pn],
                          axis=0)
    dd = jax.lax.dot_general(lhs, rhs, _DN_TT,
                             preferred_element_type=f32)
    B2 = dd[:_NG]
    C2 = dd[_NG:]
    A2 = jnp.concatenate([B2[-1:], B2[:-1]], axis=0)

    den = (A2 + B2) - 2.0 * C2
    num = A2 - C2
    num2 = num + num
    cstar = jnp.where(den > 0.0, num * pl.reciprocal(den, approx=False), 0.0)
    bc = jnp.clip(jnp.floor(cstar * 10.0 + 0.5), 0.0, 9.0) / 10.0
    bd = A2 - bc * num2 + (bc * bc) * den

    dmin = jnp.min(bd, axis=0, keepdims=True)
    oh = (bd == dmin).astype(f32)
    csel = jnp.sum(oh * bc, axis=0, keepdims=True)
    gx4 = jnp.concatenate([gxr, gxpr, gyr, gypr], axis=0)
    sel4 = jax.lax.dot_general(gx4, oh, _DN_NN,
                               preferred_element_type=f32)
    omc = 1.0 - csel
    nx = sel4[0:1] * csel + sel4[1:2] * omc
    ny = sel4[2:3] * csel + sel4[3:4] * omc
    r1 = _smooth_l1(ppxr - nx) + _smooth_l1(ppyr - ny)

    lhsp = jnp.concatenate([pn, ipxr, ipyr, ones], axis=0)
    rhsg = jnp.concatenate([ones, -2.0 * gxr, -2.0 * gyr, gg], axis=0)
    B2T = jax.lax.dot_general(lhsp, rhsg, _DN_TT,
                              preferred_element_type=f32)
    dminT = jnp.min(B2T, axis=0, keepdims=True)
    oh2 = (B2T == dminT).astype(f32)
    pp2 = jnp.concatenate([ppxr, ppyr], axis=0)
    sp = jax.lax.dot_general(pp2, oh2, _DN_NN,
                             preferred_element_type=f32)
    l2 = _smooth_l1(sp[0:1] - gxr) + _smooth_l1(sp[1:2] - gyr)
    r2 = l2 * kpmr
    return r1, r2


def _dm_kernel(ipx, ipy, ppx, ppy, gxv, gyv, kpmv, out, s1a, s2a, s3a):
    b = pl.program_id(0)

    @pl.when(b == 0)
    def _init():
        s1a[...] = jnp.zeros_like(s1a)
        s2a[...] = jnp.zeros_like(s2a)
        s3a[...] = jnp.zeros_like(s3a)

    s1 = jnp.zeros((1, _NP), jnp.float32)
    s2 = jnp.zeros((1, _NG), jnp.float32)
    s3 = jnp.zeros((1, _NG), jnp.float32)
    for i in range(_BB):
        r1, r2 = _one_instance(ipx[i], ipy[i], ppx[i], ppy[i],
                               gxv[i], gyv[i], kpmv[i])
        s1 = s1 + r1
        s2 = s2 + r2
        s3 = s3 + kpmv[i]

    s1a[...] = s1a[...] + s1
    s2a[...] = s2a[...] + s2
    s3a[...] = s3a[...] + s3

    @pl.when(b == (_B // _BB) - 1)
    def _final():
        t1 = jnp.sum(s1a[...])
        t2 = jnp.sum(s2a[...])
        t3 = jnp.sum(s3a[...])
        loss = 0.5 * (t2 / (2.0 * t3 + 1.0)
                      + t1 / np.float32(_B * _NP * 2))
        out[...] = jnp.broadcast_to(loss, (1, 1))


def _run(ipx3, ipy3, ppx3, ppy3, gx3, gy3, kpm3, interpret=False):
    row_spec = pl.BlockSpec((_BB, 1, _NP), lambda b: (b, 0, 0))
    return pl.pallas_call(
        _dm_kernel,
        grid=(_B // _BB,),
        in_specs=[row_spec] * 7,
        out_specs=pl.BlockSpec((1, 1), lambda b: (0, 0)),
        out_shape=jax.ShapeDtypeStruct((1, 1), jnp.float32),
        scratch_shapes=[pltpu.VMEM((1, _NP), jnp.float32)] * 3,
        interpret=interpret,
    )(ipx3, ipy3, ppx3, ppy3, gx3, gy3, kpm3)


def kernel(ini_pred_poly, pred_poly, gt_poly, keyPointsMask):
    ipx3 = ini_pred_poly[:, :, 0].reshape(_B, 1, _NP)
    ipy3 = ini_pred_poly[:, :, 1].reshape(_B, 1, _NP)
    ppx3 = pred_poly[:, :, 0].reshape(_B, 1, _NP)
    ppy3 = pred_poly[:, :, 1].reshape(_B, 1, _NP)
    gx3 = gt_poly[:, :, 0].reshape(_B, 1, _NG)
    gy3 = gt_poly[:, :, 1].reshape(_B, 1, _NG)
    kpm3 = keyPointsMask.reshape(_B, 1, _NG)
    out = _run(ipx3, ipy3, ppx3, ppy3, gx3, gy3, kpm3)
    return out[0, 0]


# speedup vs baseline: 5.2251x; 1.0104x over previous
import jax
import jax.numpy as jnp
import numpy as np
from jax.experimental import pallas as pl
from jax.experimental.pallas import tpu as pltpu

_B, _NP, _NG, _TIME = 256, 128, 128, 10
_BB = 128

_DN_TT = (((0,), (0,)), ((), ()))
_DN_NN = (((1,), (0,)), ((), ()))


def _smooth_l1(d):
    a = jnp.abs(d)
    return jnp.where(a < 1.0, 0.5 * a * a, a - 0.5)


def _one_instance(ipxr, ipyr, ppxr, ppyr, gxr, gyr, kpmr):
    f32 = jnp.float32
    ones = jnp.ones((1, _NP), f32)
    zeros = jnp.zeros((1, _NP), f32)
    gxpr = jnp.concatenate([gxr[:, -1:], gxr[:, :-1]], axis=1)
    gypr = jnp.concatenate([gyr[:, -1:], gyr[:, :-1]], axis=1)

    pn = ipxr * ipxr + ipyr * ipyr
    gg = gxr * gxr + gyr * gyr
    gg2 = gxpr * gxr + gypr * gyr
    hxr = 0.5 * (gxpr + gxr)
    hyr = 0.5 * (gypr + gyr)
    lhs = jnp.concatenate([
        jnp.concatenate([gxr, hxr], axis=1),
        jnp.concatenate([gyr, hyr], axis=1),
        jnp.concatenate([gg, gg2], axis=1),
        jnp.concatenate([ones, ones], axis=1),
    ], axis=0)
    rhs = jnp.concatenate([-2.0 * ipxr, -2.0 * ipyr, ones, pn],
                          axis=0)
    dd = jax.lax.dot_general(lhs, rhs, _DN_TT,
                             preferred_element_type=f32)
    B2 = dd[:_NG]
    C2 = dd[_NG:]
    A2 = jnp.concatenate([B2[-1:], B2[:-1]], axis=0)

    den = (A2 + B2) - 2.0 * C2
    num = A2 - C2
    num2 = num + num
    cstar = jnp.where(den > 0.0, num * pl.reciprocal(den, approx=False), 0.0)
    bc = jnp.clip(jnp.floor(cstar * 10.0 + 0.5), 0.0, 9.0) / 10.0
    bd = A2 - bc * num2 + (bc * bc) * den

    dmin = jnp.min(bd, axis=0, keepdims=True)
    oh = (bd == dmin).astype(f32)
    csel = jnp.sum(oh * bc, axis=0, keepdims=True)
    gx4 = jnp.concatenate([gxr, gxpr, gyr, gypr], axis=0)
    sel4 = jax.lax.dot_general(gx4, oh, _DN_NN,
                               preferred_element_type=f32)
    omc = 1.0 - csel
    nx = sel4[0:1] * csel + sel4[1:2] * omc
    ny = sel4[2:3] * csel + sel4[3:4] * omc
    r1 = _smooth_l1(ppxr - nx) + _smooth_l1(ppyr - ny)

    lhsp = jnp.concatenate([pn, ipxr, ipyr, ones], axis=0)
    rhsg = jnp.concatenate([ones, -2.0 * gxr, -2.0 * gyr, gg], axis=0)
    B2T = jax.lax.dot_general(lhsp, rhsg, _DN_TT,
                              preferred_element_type=f32)
    dminT = jnp.min(B2T, axis=0, keepdims=True)
    oh2 = (B2T == dminT).astype(f32)
    pp2 = jnp.concatenate([ppxr, ppyr], axis=0)
    sp = jax.lax.dot_general(pp2, oh2, _DN_NN,
                             preferred_element_type=f32)
    l2 = _smooth_l1(sp[0:1] - gxr) + _smooth_l1(sp[1:2] - gyr)
    r2 = l2 * kpmr
    return r1, r2


def _dm_kernel(ipx, ipy, ppx, ppy, gxv, gyv, kpmv, out, s1a, s2a, s3a):
    b = pl.program_id(0)

    @pl.when(b == 0)
    def _init():
        s1a[...] = jnp.zeros_like(s1a)
        s2a[...] = jnp.zeros_like(s2a)
        s3a[...] = jnp.zeros_like(s3a)

    s1 = jnp.zeros((1, _NP), jnp.float32)
    s2 = jnp.zeros((1, _NG), jnp.float32)
    s3 = jnp.zeros((1, _NG), jnp.float32)
    for i in range(_BB):
        r1, r2 = _one_instance(ipx[i], ipy[i], ppx[i], ppy[i],
                               gxv[i], gyv[i], kpmv[i])
        s1 = s1 + r1
        s2 = s2 + r2
        s3 = s3 + kpmv[i]

    s1a[...] = s1a[...] + s1
    s2a[...] = s2a[...] + s2
    s3a[...] = s3a[...] + s3

    @pl.when(b == (_B // _BB) - 1)
    def _final():
        t1 = jnp.sum(s1a[...])
        t2 = jnp.sum(s2a[...])
        t3 = jnp.sum(s3a[...])
        loss = 0.5 * (t2 / (2.0 * t3 + 1.0)
                      + t1 / np.float32(_B * _NP * 2))
        out[...] = jnp.broadcast_to(loss, (1, 1))


def _run(ipx3, ipy3, ppx3, ppy3, gx3, gy3, kpm3, interpret=False):
    row_spec = pl.BlockSpec((_BB, 1, _NP), lambda b: (b, 0, 0))
    return pl.pallas_call(
        _dm_kernel,
        grid=(_B // _BB,),
        in_specs=[row_spec] * 7,
        out_specs=pl.BlockSpec((1, 1), lambda b: (0, 0)),
        out_shape=jax.ShapeDtypeStruct((1, 1), jnp.float32),
        scratch_shapes=[pltpu.VMEM((1, _NP), jnp.float32)] * 3,
        interpret=interpret,
    )(ipx3, ipy3, ppx3, ppy3, gx3, gy3, kpm3)


def kernel(ini_pred_poly, pred_poly, gt_poly, keyPointsMask):
    ipx3 = ini_pred_poly[:, :, 0].reshape(_B, 1, _NP)
    ipy3 = ini_pred_poly[:, :, 1].reshape(_B, 1, _NP)
    ppx3 = pred_poly[:, :, 0].reshape(_B, 1, _NP)
    ppy3 = pred_poly[:, :, 1].reshape(_B, 1, _NP)
    gx3 = gt_poly[:, :, 0].reshape(_B, 1, _NG)
    gy3 = gt_poly[:, :, 1].reshape(_B, 1, _NG)
    kpm3 = keyPointsMask.reshape(_B, 1, _NG)
    out = _run(ipx3, ipy3, ppx3, ppy3, gx3, gy3, kpm3)
    return out[0, 0]


# speedup vs baseline: 5.9841x; 1.1453x over previous
import jax
import jax.numpy as jnp
import numpy as np
from jax.experimental import pallas as pl
from jax.experimental.pallas import tpu as pltpu

_B, _NP, _NG, _TIME = 256, 128, 128, 10
_BB = 64

_DN_TT = (((0,), (0,)), ((), ()))
_DN_NN = (((1,), (0,)), ((), ()))


def _smooth_l1(d):
    a = jnp.abs(d)
    return jnp.where(a < 1.0, 0.5 * a * a, a - 0.5)


def _one_instance(ipxr, ipyr, ppxr, ppyr, gxr, gyr, kpmr, ipxc, ipyc):
    f32 = jnp.float32
    ones = jnp.ones((1, _NP), f32)
    zeros = jnp.zeros((1, _NP), f32)
    gxpr = jnp.concatenate([gxr[:, -1:], gxr[:, :-1]], axis=1)
    gypr = jnp.concatenate([gyr[:, -1:], gyr[:, :-1]], axis=1)

    pn = ipxr * ipxr + ipyr * ipyr
    gg = gxr * gxr + gyr * gyr
    gg2 = gxpr * gxr + gypr * gyr
    hxr = 0.5 * (gxpr + gxr)
    hyr = 0.5 * (gypr + gyr)
    lhs = jnp.concatenate([
        jnp.concatenate([gxr, hxr], axis=1),
        jnp.concatenate([gyr, hyr], axis=1),
        jnp.concatenate([gg, gg2], axis=1),
        jnp.concatenate([ones, ones], axis=1),
    ], axis=0)
    rhs = jnp.concatenate([-2.0 * ipxr, -2.0 * ipyr, ones, pn],
                          axis=0)
    dd = jax.lax.dot_general(lhs, rhs, _DN_TT,
                             preferred_element_type=f32)
    B2 = dd[:_NG]
    C2 = dd[_NG:]
    A2 = jnp.concatenate([B2[-1:], B2[:-1]], axis=0)

    den = (A2 + B2) - 2.0 * C2
    num = A2 - C2
    num2 = num + num
    cstar = jnp.where(den > 0.0, num * pl.reciprocal(den, approx=False), 0.0)
    bc = jnp.clip(jnp.floor(cstar * 10.0 + 0.5), 0.0, 9.0) / 10.0
    bd = A2 - bc * num2 + (bc * bc) * den

    dmin = jnp.min(bd, axis=0, keepdims=True)
    oh = (bd == dmin).astype(f32)
    csel = jnp.sum(oh * bc, axis=0, keepdims=True)
    gx4 = jnp.concatenate([gxr, gxpr, gyr, gypr], axis=0)
    sel4 = jax.lax.dot_general(gx4, oh, _DN_NN,
                               preferred_element_type=f32)
    omc = 1.0 - csel
    nx = sel4[0:1] * csel + sel4[1:2] * omc
    ny = sel4[2:3] * csel + sel4[3:4] * omc
    r1 = _smooth_l1(ppxr - nx) + _smooth_l1(ppyr - ny)

    dxT = ipxc - gxr
    dyT = ipyc - gyr
    B2T = dxT * dxT + dyT * dyT
    dminT = jnp.min(B2T, axis=0, keepdims=True)
    oh2 = (B2T == dminT).astype(f32)
    pp2 = jnp.concatenate([ppxr, ppyr], axis=0)
    sp = jax.lax.dot_general(pp2, oh2, _DN_NN,
                             preferred_element_type=f32)
    l2 = _smooth_l1(sp[0:1] - gxr) + _smooth_l1(sp[1:2] - gyr)
    r2 = l2 * kpmr
    return r1, r2


def _dm_kernel(ipx, ipy, ppx, ppy, gxv, gyv, kpmv, ipxc, ipyc,
               out, s1a, s2a, s3a):
    b = pl.program_id(0)

    @pl.when(b == 0)
    def _init():
        s1a[...] = jnp.zeros_like(s1a)
        s2a[...] = jnp.zeros_like(s2a)
        s3a[...] = jnp.zeros_like(s3a)

    s1 = jnp.zeros((1, _NP), jnp.float32)
    s2 = jnp.zeros((1, _NG), jnp.float32)
    s3 = jnp.zeros((1, _NG), jnp.float32)
    for i in range(_BB):
        r1, r2 = _one_instance(ipx[i], ipy[i], ppx[i], ppy[i],
                               gxv[i], gyv[i], kpmv[i],
                               ipxc[i], ipyc[i])
        s1 = s1 + r1
        s2 = s2 + r2
        s3 = s3 + kpmv[i]

    s1a[...] = s1a[...] + s1
    s2a[...] = s2a[...] + s2
    s3a[...] = s3a[...] + s3

    @pl.when(b == (_B // _BB) - 1)
    def _final():
        t1 = jnp.sum(s1a[...])
        t2 = jnp.sum(s2a[...])
        t3 = jnp.sum(s3a[...])
        loss = 0.5 * (t2 / (2.0 * t3 + 1.0)
                      + t1 / np.float32(_B * _NP * 2))
        out[...] = jnp.broadcast_to(loss, (1, 1))


def _run(ipx3, ipy3, ppx3, ppy3, gx3, gy3, kpm3, ipxc3, ipyc3,
         interpret=False):
    row_spec = pl.BlockSpec((_BB, 1, _NP), lambda b: (b, 0, 0))
    col_spec = pl.BlockSpec((_BB, _NP, 1), lambda b: (b, 0, 0))
    return pl.pallas_call(
        _dm_kernel,
        grid=(_B // _BB,),
        in_specs=[row_spec] * 7 + [col_spec] * 2,
        out_specs=pl.BlockSpec((1, 1), lambda b: (0, 0)),
        out_shape=jax.ShapeDtypeStruct((1, 1), jnp.float32),
        scratch_shapes=[pltpu.VMEM((1, _NP), jnp.float32)] * 3,
        interpret=interpret,
    )(ipx3, ipy3, ppx3, ppy3, gx3, gy3, kpm3, ipxc3, ipyc3)


def kernel(ini_pred_poly, pred_poly, gt_poly, keyPointsMask):
    ipx3 = ini_pred_poly[:, :, 0].reshape(_B, 1, _NP)
    ipy3 = ini_pred_poly[:, :, 1].reshape(_B, 1, _NP)
    ppx3 = pred_poly[:, :, 0].reshape(_B, 1, _NP)
    ppy3 = pred_poly[:, :, 1].reshape(_B, 1, _NP)
    gx3 = gt_poly[:, :, 0].reshape(_B, 1, _NG)
    gy3 = gt_poly[:, :, 1].reshape(_B, 1, _NG)
    kpm3 = keyPointsMask.reshape(_B, 1, _NG)
    ipxc3 = ini_pred_poly[:, :, 0].reshape(_B, _NP, 1)
    ipyc3 = ini_pred_poly[:, :, 1].reshape(_B, _NP, 1)
    out = _run(ipx3, ipy3, ppx3, ppy3, gx3, gy3, kpm3, ipxc3, ipyc3)
    return out[0, 0]
